# baseline (speedup 1.0000x reference)
import functools

import jax
import jax.numpy as jnp
from jax import lax
from jax.experimental import pallas as pl
from jax.experimental.pallas import tpu as pltpu
from jax.experimental.pallas import tpu_sc as plsc

_HD = 64
_NH = 2
_DH = 32
_SLOTS = 6
_VOCAB = 64
_L = 2048
_B = 8
_QB = 128

_DNT = (((1,), (1,)), ((), ()))


def _dot_t(a, b):
    return lax.dot_general(a, b, _DNT, preferred_element_type=jnp.float32)


def _encoder_kernel(seq_ref, embed_ref, ipw_ref, ipb_ref,
                    aow_ref, aob_ref, w1_ref, b1_ref, w2_ref, b2_ref,
                    ln1g_ref, ln1b_ref, ln2g_ref, ln2b_ref,
                    sw_ref, sb_ref, hid_out_ref, s2f_out_ref, a_ref):
    seqcol = seq_ref[0]
    ids = lax.broadcasted_iota(jnp.int32, (_L, _VOCAB), 1)
    oneh = (ids == seqcol).astype(jnp.float32)
    h = jnp.dot(oneh, embed_ref[...], preferred_element_type=jnp.float32)
    qkv = _dot_t(h, ipw_ref[...]) + ipb_ref[0]

    inv = 1.0 / jnp.sqrt(jnp.float32(_DH))
    for hd in range(_NH):
        q = qkv[:, 32 * hd:32 * hd + 32] * inv
        k = qkv[:, 64 + 32 * hd:96 + 32 * hd]
        v = qkv[:, 128 + 32 * hd:160 + 32 * hd]
        for rb in range(_L // _QB):
            qb = q[rb * _QB:(rb + 1) * _QB]
            p = jnp.exp(_dot_t(qb, k))
            denom = jnp.sum(p, axis=-1, keepdims=True)
            o = jnp.dot(p, v, preferred_element_type=jnp.float32) / denom
            a_ref[rb * _QB:(rb + 1) * _QB, 32 * hd:32 * hd + 32] = o

    a = _dot_t(a_ref[...], aow_ref[...]) + aob_ref[0]
    x = h + a
    mu = jnp.mean(x, axis=-1, keepdims=True)
    xc = x - mu
    var = jnp.mean(xc * xc, axis=-1, keepdims=True)
    h1 = xc / jnp.sqrt(var + 1e-5) * ln1g_ref[0] + ln1b_ref[0]
    ff = jnp.maximum(_dot_t(h1, w1_ref[...]) + b1_ref[0], 0.0)
    ff = _dot_t(ff, w2_ref[...]) + b2_ref[0]
    x2 = h1 + ff
    mu2 = jnp.mean(x2, axis=-1, keepdims=True)
    xc2 = x2 - mu2
    var2 = jnp.mean(xc2 * xc2, axis=-1, keepdims=True)
    hidden = xc2 / jnp.sqrt(var2 + 1e-5) * ln2g_ref[0] + ln2b_ref[0]
    hid_out_ref[0, :, 0:_HD] = hidden

    sl = _dot_t(hidden, sw_ref[...])
    logit1 = sl[:, 0:1] + sb_ref[0, 0]
    logit2 = sl[:, 1:2] + sb_ref[0, 1]
    keep = (logit1 > 0.0).astype(jnp.float32)
    s2f_out_ref[0] = jax.nn.sigmoid(logit2) * keep


_BIG = jnp.int32(1 << 30)
_CH = _L // 4


_GDN = lax.GatherDimensionNumbers(
    offset_dims=(), collapsed_slice_dims=(0,), start_index_map=(0,))


def _lane_shuffle(x, perm):
    return lax.gather(x, perm[:, None], _GDN, slice_sizes=(1,),
                      mode=lax.GatherScatterMode.PROMISE_IN_BOUNDS)


def _all_max(x, lane):
    for k in (1, 2, 4, 8):
        x = jnp.maximum(x, _lane_shuffle(x, lane ^ k))
    return x


def _all_min(x, lane):
    for k in (1, 2, 4, 8):
        x = jnp.minimum(x, _lane_shuffle(x, lane ^ k))
    return x


def _sc_topk(s2f_hbm, hid_hbm, mem_hbm, candv_hbm, candi_hbm,
             chunk_v, idx_v, selv_v, rows_v, cv_v, ci_v, sem):
    c = lax.axis_index("c")
    s = lax.axis_index("s")
    batch = c * 4 + s // 4
    qt = s % 4
    base = batch * _L + qt * _CH
    lane = lax.iota(jnp.int32, 16)

    pltpu.sync_copy(s2f_hbm.at[pl.ds(base, _CH)], chunk_v)

    nvec = _CH // 16
    selv = jnp.full((16,), -1.0, jnp.float32)
    seli = jnp.full((16,), _BIG, jnp.int32)
    for i in range(_SLOTS):
        m = jnp.full((16,), -1.0, jnp.float32)
        for j in range(nvec):
            m = jnp.maximum(m, chunk_v[pl.ds(j * 16, 16)])
        mval = _all_max(m, lane)
        best = jnp.full((16,), _BIG, jnp.int32)
        for j in range(nvec):
            cj = chunk_v[pl.ds(j * 16, 16)]
            gidx = lane + (base + j * 16)
            best = jnp.minimum(best, jnp.where(cj == mval, gidx, _BIG))
        jsel = _all_min(best, lane)
        selv = jnp.where(lane == i, mval, selv)
        seli = jnp.where(lane == i, jsel, seli)
        for j in range(nvec):
            gidx = lane + (base + j * 16)
            cj = chunk_v[pl.ds(j * 16, 16)]
            chunk_v[pl.ds(j * 16, 16)] = jnp.where(gidx == jsel, -1.0, cj)

    idx_v[...] = seli
    selv_v[...] = selv
    w_off = (batch * 4 + qt) * 16
    pltpu.sync_copy(selv_v, candv_hbm.at[pl.ds(w_off, 16)])
    pltpu.sync_copy(idx_v, candi_hbm.at[pl.ds(w_off, 16)])
    plsc.subcore_barrier()

    @pl.when(qt == 0)
    def _():
        pltpu.sync_copy(candv_hbm.at[pl.ds(batch * 64, 64)], cv_v)
        pltpu.sync_copy(candi_hbm.at[pl.ds(batch * 64, 64)], ci_v)
        selidx = jnp.full((16,), jnp.int32(0), jnp.int32)
        first = jnp.full((16,), jnp.int32(0), jnp.int32)
        for i in range(_SLOTS):
            m = jnp.full((16,), -1.0, jnp.float32)
            for j in range(4):
                m = jnp.maximum(m, cv_v[pl.ds(j * 16, 16)])
            mval = _all_max(m, lane)
            best = jnp.full((16,), _BIG, jnp.int32)
            for j in range(4):
                cvj = cv_v[pl.ds(j * 16, 16)]
                cij = ci_v[pl.ds(j * 16, 16)]
                best = jnp.minimum(best, jnp.where(cvj == mval, cij, _BIG))
            jsel = _all_min(best, lane)
            if i == 0:
                first = jsel
            selidx = jnp.where(lane == i, jsel, selidx)
            for j in range(4):
                cvj = cv_v[pl.ds(j * 16, 16)]
                cij = ci_v[pl.ds(j * 16, 16)]
                cv_v[pl.ds(j * 16, 16)] = jnp.where(cij == jsel, -1.0, cvj)
        selidx = jnp.where(lane < _SLOTS, selidx, first)
        idx_v[...] = selidx
        pltpu.async_copy(hid_hbm.at[idx_v], rows_v, sem).wait()
        pltpu.sync_copy(rows_v.at[pl.ds(0, 8)], mem_hbm.at[batch])


def _run_sc_topk(s2f_flat, hid_flat):
    mesh = plsc.VectorSubcoreMesh(core_axis_name="c", subcore_axis_name="s")
    f = functools.partial(
        pl.kernel,
        mesh=mesh,
        out_type=(
            jax.ShapeDtypeStruct((_B, 8, 128), jnp.float32),
            jax.ShapeDtypeStruct((_B * 4 * 16,), jnp.float32),
            jax.ShapeDtypeStruct((_B * 4 * 16,), jnp.int32),
        ),
        scratch_types=[
            pltpu.VMEM((_CH,), jnp.float32),
            pltpu.VMEM((16,), jnp.int32),
            pltpu.VMEM((16,), jnp.float32),
            pltpu.VMEM((16, 128), jnp.float32),
            pltpu.VMEM((64,), jnp.float32),
            pltpu.VMEM((64,), jnp.int32),
            pltpu.SemaphoreType.DMA,
        ],
    )(_sc_topk)
    mem, _, _ = f(s2f_flat, hid_flat)
    return mem


def _reader_kernel(mem_ref, query_ref, target_ref, qemb_ref, qpw_ref, qpb_ref,
                   rdw_ref, rdb_ref, out_ref):
    mem = mem_ref[...][:, :, 0:_HD]
    voc = lax.broadcasted_iota(jnp.int32, (_B, _VOCAB), 1)
    qoh = (voc == query_ref[...]).astype(jnp.float32)
    q_h = jnp.dot(qoh, qemb_ref[...], preferred_element_type=jnp.float32)
    qq = _dot_t(q_h, qpw_ref[...]) + qpb_ref[0]
    rs = jnp.sum(mem * qq[:, None, :], axis=-1) / jnp.sqrt(jnp.float32(_HD))
    slot = lax.broadcasted_iota(jnp.int32, (_B, 8), 1)
    rs = jnp.where(slot < _SLOTS, rs, -1e30)
    mx = jnp.max(rs, axis=1, keepdims=True)
    e = jnp.where(slot < _SLOTS, jnp.exp(rs - mx), 0.0)
    wts = e / jnp.sum(e, axis=1, keepdims=True)
    pooled = jnp.sum(mem * wts[:, :, None], axis=1)
    logits = _dot_t(pooled, rdw_ref[...]) + rdb_ref[0]
    lmx = jnp.max(logits, axis=1, keepdims=True)
    lse = jnp.log(jnp.sum(jnp.exp(logits - lmx), axis=1, keepdims=True)) + lmx
    toh = (voc == target_ref[...]).astype(jnp.float32)
    tlogit = jnp.sum(toh * logits, axis=1, keepdims=True)
    loss = jnp.mean(lse - tlogit)
    out_ref[...] = jnp.reshape(loss, (1, 1))


def kernel(seq, query, target, embed, in_proj_w, in_proj_b, attn_out_w, attn_out_b,
           ff_w1, ff_b1, ff_w2, ff_b2, ln1_g, ln1_b, ln2_g, ln2_b,
           s1_w, s1_b, s2_w, s2_b, qp_w, qp_b, rd_out_w, rd_out_b, qembed):
    seq_c = seq.astype(jnp.int32).reshape(_B, _L, 1)
    sw = jnp.concatenate([s1_w, s2_w], axis=0)
    sb = jnp.concatenate([s1_b, s2_b], axis=0).reshape(1, 2)

    def row(v):
        return v.reshape(1, -1)

    full = lambda shape: pl.BlockSpec(shape, lambda b: (0,) * len(shape))
    hid, s2f = pl.pallas_call(
        _encoder_kernel,
        grid=(_B,),
        in_specs=[
            pl.BlockSpec((1, _L, 1), lambda b: (b, 0, 0)),
            full((_VOCAB, _HD)),
            full((3 * _HD, _HD)), full((1, 3 * _HD)),
            full((_HD, _HD)), full((1, _HD)),
            full((2 * _HD, _HD)), full((1, 2 * _HD)),
            full((_HD, 2 * _HD)), full((1, _HD)),
            full((1, _HD)), full((1, _HD)),
            full((1, _HD)), full((1, _HD)),
            full((2, _HD)), full((1, 2)),
        ],
        out_specs=[pl.BlockSpec((1, _L, 128), lambda b: (b, 0, 0)),
                   pl.BlockSpec((1, _L, 1), lambda b: (b, 0, 0))],
        out_shape=[jax.ShapeDtypeStruct((_B, _L, 128), jnp.float32),
                   jax.ShapeDtypeStruct((_B, _L, 1), jnp.float32)],
        scratch_shapes=[pltpu.VMEM((_L, _HD), jnp.float32)],
    )(
        seq_c, embed, in_proj_w, row(in_proj_b), attn_out_w, row(attn_out_b),
        ff_w1, row(ff_b1), ff_w2, row(ff_b2),
        row(ln1_g), row(ln1_b), row(ln2_g), row(ln2_b), sw, sb,
    )

    mem = _run_sc_topk(s2f.reshape(_B * _L), hid.reshape(_B * _L, 128))

    out = pl.pallas_call(
        _reader_kernel,
        out_shape=jax.ShapeDtypeStruct((1, 1), jnp.float32),
    )(
        mem, query.astype(jnp.int32).reshape(_B, 1),
        target.astype(jnp.int32).reshape(_B, 1),
        qembed, qp_w, row(qp_b), rd_out_w, row(rd_out_b),
    )
    return out[0, 0]
